# baseline (speedup 1.0000x reference)
import functools

import jax
import jax.numpy as jnp
from jax import lax
from jax.experimental import pallas as pl
from jax.experimental.pallas import tpu as pltpu
from jax.experimental.pallas import tpu_sc as plsc

ROWS = 128
VOCAB = 100000
NC = 2
NS = 16
L = 16
RB = 8
NRB = ROWS // RB
CHUNKC = 3200
NCHUNKS = 16
HALF0 = 49920
TAILC = VOCAB - 781 * 128
TAIL_START = 781 * 128
UNROLL = 4
STEPS = CHUNKC // (L * UNROLL)

_cache = {}


def _neg_log_noise():
    if "g" not in _cache:
        with jax.default_device(jax.devices("cpu")[0]):
            noise = jax.random.exponential(
                jax.random.key(42), (ROWS, VOCAB), dtype=jnp.float32)
            noise = jnp.clip(noise, 1e-10, None)
            _cache["g"] = jax.device_get(-jnp.log(noise))
    return _cache["g"]


def _take(vm, va, bs, bi):
    t = (bs > vm) | ((bs == vm) & (bi < va))
    return jnp.where(t, bs, vm), jnp.where(t, bi, va)


def _build_sampler():
    mesh = plsc.VectorSubcoreMesh(core_axis_name="c", subcore_axis_name="s")

    @functools.partial(
        pl.kernel,
        out_type=jax.ShapeDtypeStruct((NRB * L,), jnp.int32),
        mesh=mesh,
        scratch_types=[
            pltpu.VMEM((ROWS * L,), jnp.float32),
            pltpu.VMEM((RB, CHUNKC), jnp.float32),
            pltpu.VMEM((RB, CHUNKC), jnp.float32),
            pltpu.VMEM((RB, CHUNKC), jnp.float32),
            pltpu.VMEM((RB, CHUNKC), jnp.float32),
            pltpu.VMEM((RB, TAILC), jnp.float32),
            pltpu.VMEM((RB, TAILC), jnp.float32),
            pltpu.VMEM((L,), jnp.float32),
            pltpu.VMEM((L,), jnp.int32),
            pltpu.VMEM((L,), jnp.float32),
            pltpu.VMEM((L,), jnp.int32),
            pltpu.VMEM((L,), jnp.int32),
            pltpu.VMEM((3 * L,), jnp.float32),
            pltpu.VMEM((3 * L,), jnp.int32),
            pltpu.VMEM_SHARED((NS * L,), jnp.float32),
            pltpu.VMEM_SHARED((NS * L,), jnp.int32),
            pltpu.SemaphoreType.DMA,
            pltpu.SemaphoreType.DMA,
        ],
    )
    def sampler(logits_hbm, gum_hbm, temps_hbm, out_hbm,
                temps_v, bx0, bx1, bg0, bg1, tbx, tbg,
                mv_v, mi_v, pv_v, pi_v, out_v, shf_s, shf_i,
                spval, spidx, sem0, sem1):
        bufx = (bx0, bx1)
        bufg = (bg0, bg1)
        sems = (sem0, sem1)
        c_ax = lax.axis_index("c")
        s_ax = lax.axis_index("s")
        rb = c_ax * (NS // 2) + s_ax // 2
        half = s_ax % 2
        r0 = rb * RB
        pltpu.sync_copy(temps_hbm, temps_v)
        lanes = lax.iota(jnp.int32, L)
        neg_inf = jnp.full((L,), -3.0e38, jnp.float32)
        big_idx = jnp.full((L,), 2**31 - 1, jnp.int32)
        shf_s[pl.ds(0, L)] = neg_inf
        shf_s[pl.ds(2 * L, L)] = neg_inf
        shf_i[pl.ds(0, L)] = big_idx
        shf_i[pl.ds(2 * L, L)] = big_idx

        tvecs = [temps_v[pl.ds((r0 + j) * L, L)] for j in range(RB)]

        def chunk_start(c):
            return jnp.where(
                c < NCHUNKS - 1,
                half * HALF0 + c * CHUNKC,
                (NCHUNKS - 1) * CHUNKC + half * (TAIL_START - NCHUNKS * CHUNKC))

        def start_dma(c, b):
            st = chunk_start(c)
            pltpu.async_copy(
                logits_hbm.at[pl.ds(r0, RB), pl.ds(st, CHUNKC)],
                bufx[b], sems[b])
            pltpu.async_copy(
                gum_hbm.at[pl.ds(r0, RB), pl.ds(st, CHUNKC)],
                bufg[b], sems[b])

        def drain(b):
            pltpu.make_async_copy(
                logits_hbm.at[pl.ds(0, RB), pl.ds(0, CHUNKC)],
                bufx[b], sems[b]).wait()
            pltpu.make_async_copy(
                gum_hbm.at[pl.ds(0, RB), pl.ds(0, CHUNKC)],
                bufg[b], sems[b]).wait()

        def process(c, b, accs):
            cbase = chunk_start(c)
            new = []
            for j in range(RB):
                slots = ((accs[j][0], accs[j][1]),) + tuple(
                    (jnp.full((L,), -3.0e38, jnp.float32),
                     jnp.zeros((L,), jnp.int32)) for _ in range(UNROLL - 1))

                def step(i, carry, _j=j, _tv=tvecs[j]):
                    out = []
                    ib = i * (UNROLL * L)
                    for u, (vm, va) in enumerate(carry):
                        off = ib + u * L
                        x = bufx[b][_j, pl.ds(off, L)]
                        g = bufg[b][_j, pl.ds(off, L)]
                        s = x + _tv * g
                        idxv = lanes + (cbase + off)
                        m = s > vm
                        out.append((jnp.maximum(s, vm),
                                    jnp.where(m, idxv, va)))
                    return tuple(out)

                slots = lax.fori_loop(0, STEPS, step, slots)
                vm, va = slots[0]
                for bs, bi in slots[1:]:
                    vm, va = _take(vm, va, bs, bi)
                new.append((vm, va))
            return tuple(new)

        accs = tuple(
            (jnp.full((L,), -3.0e38, jnp.float32),
             jnp.zeros((L,), jnp.int32)) for _ in range(RB))
        start_dma(0, 0)

        def pair_body(cc, accs):
            c0 = 2 * cc
            start_dma(c0 + 1, 1)
            drain(0)
            accs = process(c0, 0, accs)

            @pl.when(c0 + 2 < NCHUNKS)
            def _():
                start_dma(c0 + 2, 0)

            drain(1)
            return process(c0 + 1, 1, accs)

        accs = lax.fori_loop(0, NCHUNKS // 2, pair_body, accs)
        accs = list(accs)

        pltpu.sync_copy(
            logits_hbm.at[pl.ds(r0, RB), pl.ds(TAIL_START, TAILC)], tbx)
        pltpu.sync_copy(
            gum_hbm.at[pl.ds(r0, RB), pl.ds(TAIL_START, TAILC)], tbg)
        for j in range(RB):
            vm, va = accs[j]
            for u in range(TAILC // L):
                x = tbx[j, pl.ds(u * L, L)]
                g = tbg[j, pl.ds(u * L, L)]
                s = x + tvecs[j] * g
                idxv = lanes + (TAIL_START + u * L)
                vm, va = _take(vm, va, s, idxv)
            accs[j] = (vm, va)

        mval = neg_inf
        midx = jnp.zeros((L,), jnp.int32)
        for j in range(RB):
            vm, va = accs[j]
            for k in (8, 4, 2, 1):
                shf_s[pl.ds(L, L)] = vm
                shf_i[pl.ds(L, L)] = va
                for off in (L + k, L - k):
                    bs = shf_s[pl.ds(off, L)]
                    bi = shf_i[pl.ds(off, L)]
                    vm, va = _take(vm, va, bs, bi)
            sel = lanes == j
            mval = jnp.where(sel, vm, mval)
            midx = jnp.where(sel, va, midx)

        mv_v[...] = mval
        mi_v[...] = midx
        pltpu.sync_copy(mv_v, spval.at[pl.ds(s_ax * L, L)])
        pltpu.sync_copy(mi_v, spidx.at[pl.ds(s_ax * L, L)])
        plsc.subcore_barrier()
        partner = s_ax ^ 1
        pltpu.sync_copy(spval.at[pl.ds(partner * L, L)], pv_v)
        pltpu.sync_copy(spidx.at[pl.ds(partner * L, L)], pi_v)
        vm, va = _take(mval, midx, pv_v[...], pi_v[...])
        out_v[...] = va
        pltpu.sync_copy(out_v, out_hbm.at[pl.ds(rb * L, L)])

    return sampler


def kernel(logits, temperatures):
    if "sampler" not in _cache:
        _cache["sampler"] = _build_sampler()
    g = jnp.asarray(_neg_log_noise())
    packed = _cache["sampler"](
        logits, g, jnp.repeat(temperatures.astype(jnp.float32), L))
    return packed.reshape(NRB, L)[:, :RB].reshape(ROWS)


# speedup vs baseline: 2.2752x; 2.2752x over previous
import functools

import jax
import jax.numpy as jnp
from jax import lax
from jax.experimental import pallas as pl
from jax.experimental.pallas import tpu as pltpu
from jax.experimental.pallas import tpu_sc as plsc

ROWS = 128
VOCAB = 100000
NC = 2
NS = 16
L = 16
RB = 8
NRB = ROWS // RB
CHUNKC = 3200
NCHUNKS = 16
HALF0 = 49920
TAILC = VOCAB - 781 * 128
TAIL_START = 781 * 128
UNROLL = 4
STEPS = CHUNKC // (L * UNROLL)

_cache = {}

CHUNKF = RB * CHUNKC
BLOCKS_PER_WORKER = NCHUNKS
TAIL0 = NRB * 2 * BLOCKS_PER_WORKER * CHUNKF
TAILF = RB * TAILC
GSIZE = TAIL0 + NRB * TAILF


def _py_chunk_start(half, c):
    if c < NCHUNKS - 1:
        return half * HALF0 + c * CHUNKC
    return (NCHUNKS - 1) * CHUNKC + half * (TAIL_START - NCHUNKS * CHUNKC)


def _neg_log_noise():
    if "g" not in _cache:
        import numpy as np
        with jax.ensure_compile_time_eval(), \
                jax.default_device(jax.devices("cpu")[0]):
            noise = jax.random.exponential(
                jax.random.key(42), (ROWS, VOCAB), dtype=jnp.float32)
            noise = jnp.clip(noise, 1e-10, None)
            g2 = np.asarray(jax.device_get(-jnp.log(noise)))
        gre = np.empty((GSIZE,), np.float32)
        for rb in range(NRB):
            rows = slice(rb * RB, rb * RB + RB)
            for half in range(2):
                for c in range(NCHUNKS):
                    st = _py_chunk_start(half, c)
                    off = ((rb * 2 + half) * BLOCKS_PER_WORKER + c) * CHUNKF
                    gre[off:off + CHUNKF] = g2[rows, st:st + CHUNKC].ravel()
            toff = TAIL0 + rb * TAILF
            gre[toff:toff + TAILF] = g2[rows, TAIL_START:VOCAB].ravel()
        _cache["g"] = gre
    return _cache["g"]


def _take(vm, va, bs, bi):
    t = (bs > vm) | ((bs == vm) & (bi < va))
    return jnp.where(t, bs, vm), jnp.where(t, bi, va)


def _build_sampler():
    mesh = plsc.VectorSubcoreMesh(core_axis_name="c", subcore_axis_name="s")

    @functools.partial(
        pl.kernel,
        out_type=jax.ShapeDtypeStruct((NRB * L,), jnp.int32),
        mesh=mesh,
        scratch_types=[
            pltpu.VMEM((ROWS * L,), jnp.float32),
            pltpu.VMEM((RB, CHUNKC), jnp.float32),
            pltpu.VMEM((RB, CHUNKC), jnp.float32),
            pltpu.VMEM((CHUNKF,), jnp.float32),
            pltpu.VMEM((CHUNKF,), jnp.float32),
            pltpu.VMEM((RB, TAILC), jnp.float32),
            pltpu.VMEM((TAILF,), jnp.float32),
            pltpu.VMEM((L,), jnp.float32),
            pltpu.VMEM((L,), jnp.int32),
            pltpu.VMEM((L,), jnp.float32),
            pltpu.VMEM((L,), jnp.int32),
            pltpu.VMEM((L,), jnp.int32),
            pltpu.VMEM((3 * L,), jnp.float32),
            pltpu.VMEM((3 * L,), jnp.int32),
            pltpu.VMEM_SHARED((NS * L,), jnp.float32),
            pltpu.VMEM_SHARED((NS * L,), jnp.int32),
            pltpu.SemaphoreType.DMA,
            pltpu.SemaphoreType.DMA,
        ],
    )
    def sampler(logits_hbm, gum_hbm, temps_hbm, out_hbm,
                temps_v, bx0, bx1, bg0, bg1, tbx, tbg,
                mv_v, mi_v, pv_v, pi_v, out_v, shf_s, shf_i,
                spval, spidx, sem0, sem1):
        bufx = (bx0, bx1)
        bufg = (bg0, bg1)
        sems = (sem0, sem1)
        c_ax = lax.axis_index("c")
        s_ax = lax.axis_index("s")
        rb = c_ax * (NS // 2) + s_ax // 2
        half = s_ax % 2
        r0 = rb * RB
        pltpu.sync_copy(temps_hbm, temps_v)
        lanes = lax.iota(jnp.int32, L)
        neg_inf = jnp.full((L,), -3.0e38, jnp.float32)
        big_idx = jnp.full((L,), 2**31 - 1, jnp.int32)
        shf_s[pl.ds(0, L)] = neg_inf
        shf_s[pl.ds(2 * L, L)] = neg_inf
        shf_i[pl.ds(0, L)] = big_idx
        shf_i[pl.ds(2 * L, L)] = big_idx

        tvecs = [temps_v[pl.ds((r0 + j) * L, L)] for j in range(RB)]

        def chunk_start(c):
            return jnp.where(
                c < NCHUNKS - 1,
                half * HALF0 + c * CHUNKC,
                (NCHUNKS - 1) * CHUNKC + half * (TAIL_START - NCHUNKS * CHUNKC))

        gbase = (rb * 2 + half) * BLOCKS_PER_WORKER * CHUNKF

        def start_dma(c, b):
            st = chunk_start(c)
            pltpu.async_copy(
                logits_hbm.at[pl.ds(r0, RB), pl.ds(st, CHUNKC)],
                bufx[b], sems[b])
            pltpu.async_copy(
                gum_hbm.at[pl.ds(gbase + c * CHUNKF, CHUNKF)],
                bufg[b], sems[b])

        def drain(b):
            pltpu.make_async_copy(
                logits_hbm.at[pl.ds(0, RB), pl.ds(0, CHUNKC)],
                bufx[b], sems[b]).wait()
            pltpu.make_async_copy(
                gum_hbm.at[pl.ds(0, CHUNKF)],
                bufg[b], sems[b]).wait()

        def process(c, b, accs):
            cbase = chunk_start(c)
            new = []
            for j in range(RB):
                slots = ((accs[j][0], accs[j][1]),) + tuple(
                    (jnp.full((L,), -3.0e38, jnp.float32),
                     jnp.zeros((L,), jnp.int32)) for _ in range(UNROLL - 1))

                def step(i, carry, _j=j, _tv=tvecs[j]):
                    out = []
                    ib = i * (UNROLL * L)
                    for u, (vm, va) in enumerate(carry):
                        off = ib + u * L
                        x = bufx[b][_j, pl.ds(off, L)]
                        g = bufg[b][pl.ds(_j * CHUNKC + off, L)]
                        s = x + _tv * g
                        idxv = lanes + (cbase + off)
                        m = s > vm
                        out.append((jnp.maximum(s, vm),
                                    jnp.where(m, idxv, va)))
                    return tuple(out)

                slots = lax.fori_loop(0, STEPS, step, slots)
                vm, va = slots[0]
                for bs, bi in slots[1:]:
                    vm, va = _take(vm, va, bs, bi)
                new.append((vm, va))
            return tuple(new)

        accs = tuple(
            (jnp.full((L,), -3.0e38, jnp.float32),
             jnp.zeros((L,), jnp.int32)) for _ in range(RB))
        start_dma(0, 0)

        def pair_body(cc, accs):
            c0 = 2 * cc
            start_dma(c0 + 1, 1)
            drain(0)
            accs = process(c0, 0, accs)

            @pl.when(c0 + 2 < NCHUNKS)
            def _():
                start_dma(c0 + 2, 0)

            drain(1)
            return process(c0 + 1, 1, accs)

        accs = lax.fori_loop(0, NCHUNKS // 2, pair_body, accs)
        accs = list(accs)

        pltpu.sync_copy(
            logits_hbm.at[pl.ds(r0, RB), pl.ds(TAIL_START, TAILC)], tbx)
        pltpu.sync_copy(
            gum_hbm.at[pl.ds(TAIL0 + rb * TAILF, TAILF)], tbg)
        for j in range(RB):
            vm, va = accs[j]
            for u in range(TAILC // L):
                x = tbx[j, pl.ds(u * L, L)]
                g = tbg[pl.ds(j * TAILC + u * L, L)]
                s = x + tvecs[j] * g
                idxv = lanes + (TAIL_START + u * L)
                vm, va = _take(vm, va, s, idxv)
            accs[j] = (vm, va)

        mval = neg_inf
        midx = jnp.zeros((L,), jnp.int32)
        for j in range(RB):
            vm, va = accs[j]
            for k in (8, 4, 2, 1):
                shf_s[pl.ds(L, L)] = vm
                shf_i[pl.ds(L, L)] = va
                for off in (L + k, L - k):
                    bs = shf_s[pl.ds(off, L)]
                    bi = shf_i[pl.ds(off, L)]
                    vm, va = _take(vm, va, bs, bi)
            sel = lanes == j
            mval = jnp.where(sel, vm, mval)
            midx = jnp.where(sel, va, midx)

        mv_v[...] = mval
        mi_v[...] = midx
        pltpu.sync_copy(mv_v, spval.at[pl.ds(s_ax * L, L)])
        pltpu.sync_copy(mi_v, spidx.at[pl.ds(s_ax * L, L)])
        plsc.subcore_barrier()
        partner = s_ax ^ 1
        pltpu.sync_copy(spval.at[pl.ds(partner * L, L)], pv_v)
        pltpu.sync_copy(spidx.at[pl.ds(partner * L, L)], pi_v)
        vm, va = _take(mval, midx, pv_v[...], pi_v[...])
        out_v[...] = va
        pltpu.sync_copy(out_v, out_hbm.at[pl.ds(rb * L, L)])

    return sampler


def kernel(logits, temperatures):
    if "sampler" not in _cache:
        _cache["sampler"] = _build_sampler()
    g = jnp.asarray(_neg_log_noise())
    packed = _cache["sampler"](
        logits, g, jnp.repeat(temperatures.astype(jnp.float32), L))
    return packed.reshape(NRB, L)[:, :RB].reshape(ROWS)


# speedup vs baseline: 3.4641x; 1.5225x over previous
import functools

import jax
import jax.numpy as jnp
from jax import lax
from jax.experimental import pallas as pl
from jax.experimental.pallas import tpu as pltpu
from jax.experimental.pallas import tpu_sc as plsc

ROWS = 128
VOCAB = 100000
NC = 2
NS = 16
NW = NC * NS
L = 16
CB = ROWS // L
VC = 128
NCHUNKS = 25
STRIPE = VC * NCHUNKS
CHUNKF = VC * ROWS
GSIZE = NW * NCHUNKS * CHUNKF
LAST_START = VOCAB - STRIPE

_cache = {}


def _stripe_start(w):
    return ((w * (LAST_START // 8)) // (NW - 1)) * 8


def _neg_log_noise():
    if "g" not in _cache:
        import numpy as np
        with jax.ensure_compile_time_eval(), \
                jax.default_device(jax.devices("cpu")[0]):
            noise = jax.random.exponential(
                jax.random.key(42), (ROWS, VOCAB), dtype=jnp.float32)
            noise = jnp.clip(noise, 1e-10, None)
            g2 = np.asarray(jax.device_get(-jnp.log(noise)))
        gt = np.ascontiguousarray(g2.T)
        gre = np.empty((GSIZE,), np.float32)
        for w in range(NW):
            sw = _stripe_start(w)
            for c in range(NCHUNKS):
                off = (w * NCHUNKS + c) * CHUNKF
                v0 = sw + c * VC
                gre[off:off + CHUNKF] = gt[v0:v0 + VC].ravel()
        _cache["g"] = gre
    return _cache["g"]


def _take(vm, va, bs, bi):
    t = (bs > vm) | ((bs == vm) & (bi < va))
    return jnp.where(t, bs, vm), jnp.where(t, bi, va)


def _build_sampler():
    mesh = plsc.VectorSubcoreMesh(core_axis_name="c", subcore_axis_name="s")

    @functools.partial(
        pl.kernel,
        out_type=(jax.ShapeDtypeStruct((NC * ROWS,), jnp.float32),
                  jax.ShapeDtypeStruct((NC * ROWS,), jnp.int32)),
        mesh=mesh,
        scratch_types=[
            pltpu.VMEM((ROWS,), jnp.float32),
            pltpu.VMEM((VC, ROWS), jnp.float32),
            pltpu.VMEM((VC, ROWS), jnp.float32),
            pltpu.VMEM((CHUNKF,), jnp.float32),
            pltpu.VMEM((CHUNKF,), jnp.float32),
            pltpu.VMEM((ROWS,), jnp.float32),
            pltpu.VMEM((ROWS,), jnp.int32),
            pltpu.VMEM((NS * ROWS,), jnp.float32),
            pltpu.VMEM((NS * ROWS,), jnp.int32),
            pltpu.VMEM((L,), jnp.float32),
            pltpu.VMEM((L,), jnp.int32),
            pltpu.VMEM_SHARED((NS * ROWS,), jnp.float32),
            pltpu.VMEM_SHARED((NS * ROWS,), jnp.int32),
            pltpu.SemaphoreType.DMA,
            pltpu.SemaphoreType.DMA,
        ],
    )
    def sampler(logits_hbm, gum_hbm, temps_hbm, outv_hbm, outi_hbm,
                temps_v, bx0, bx1, bg0, bg1, mvals_v, midx_v,
                allv_v, alli_v, ov_v, oi_v, spval, spidx, sem0, sem1):
        bufx = (bx0, bx1)
        bufg = (bg0, bg1)
        sems = (sem0, sem1)
        c_ax = lax.axis_index("c")
        s_ax = lax.axis_index("s")
        w = c_ax * NS + s_ax
        sw = _stripe_start(w)
        gbase = w * NCHUNKS * CHUNKF
        pltpu.sync_copy(temps_hbm, temps_v)
        tvecs = [temps_v[pl.ds(cb * L, L)] for cb in range(CB)]

        def start_dma(c, b):
            pltpu.async_copy(
                logits_hbm.at[pl.ds(sw + c * VC, VC), :],
                bufx[b], sems[b])
            pltpu.async_copy(
                gum_hbm.at[pl.ds(gbase + c * CHUNKF, CHUNKF)],
                bufg[b], sems[b])

        def drain(b):
            pltpu.make_async_copy(
                logits_hbm.at[pl.ds(0, VC), :], bufx[b], sems[b]).wait()
            pltpu.make_async_copy(
                gum_hbm.at[pl.ds(0, CHUNKF)], bufg[b], sems[b]).wait()

        def process(c, b, accs):
            vbase = sw + c * VC

            def step2(i, carry):
                out = []
                idxv = jnp.full((L,), vbase + i, jnp.int32)
                goff = i * ROWS
                for cb, (vm, va) in enumerate(carry):
                    x = bufx[b][i, pl.ds(cb * L, L)]
                    g = bufg[b][pl.ds(goff + cb * L, L)]
                    s = x + tvecs[cb] * g
                    m = s > vm
                    out.append((jnp.maximum(s, vm),
                                jnp.where(m, idxv, va)))
                return tuple(out)

            return lax.fori_loop(0, VC, step2, accs)

        accs = tuple(
            (jnp.full((L,), -3.0e38, jnp.float32),
             jnp.zeros((L,), jnp.int32)) for _ in range(CB))
        start_dma(0, 0)

        def pair_body(cc, accs):
            c0 = 2 * cc
            start_dma(c0 + 1, 1)
            drain(0)
            accs = process(c0, 0, accs)

            @pl.when(c0 + 2 < NCHUNKS)
            def _():
                start_dma(c0 + 2, 0)

            drain(1)
            return process(c0 + 1, 1, accs)

        accs = lax.fori_loop(0, NCHUNKS // 2, pair_body, accs)
        drain(0)
        accs = process(NCHUNKS - 1, 0, accs)

        for cb in range(CB):
            vm, va = accs[cb]
            mvals_v[pl.ds(cb * L, L)] = vm
            midx_v[pl.ds(cb * L, L)] = va
        pltpu.sync_copy(mvals_v, spval.at[pl.ds(s_ax * ROWS, ROWS)])
        pltpu.sync_copy(midx_v, spidx.at[pl.ds(s_ax * ROWS, ROWS)])
        plsc.subcore_barrier()

        pltpu.sync_copy(spval, allv_v)
        pltpu.sync_copy(spidx, alli_v)
        mcb = s_ax % CB
        vm = jnp.full((L,), -3.0e38, jnp.float32)
        va = jnp.zeros((L,), jnp.int32)
        for t in range(NS):
            bs = allv_v[pl.ds(t * ROWS + mcb * L, L)]
            bi = alli_v[pl.ds(t * ROWS + mcb * L, L)]
            vm, va = _take(vm, va, bs, bi)
        ov_v[...] = vm
        oi_v[...] = va
        obase = c_ax * ROWS + mcb * L
        pltpu.sync_copy(ov_v, outv_hbm.at[pl.ds(obase, L)])
        pltpu.sync_copy(oi_v, outi_hbm.at[pl.ds(obase, L)])

    return sampler


def kernel(logits, temperatures):
    if "sampler" not in _cache:
        _cache["sampler"] = _build_sampler()
    g = jnp.asarray(_neg_log_noise())
    vals, idxs = _cache["sampler"](
        logits.T, g, temperatures.astype(jnp.float32))
    v = vals.reshape(NC, ROWS)
    i = idxs.reshape(NC, ROWS)
    take = (v[1] > v[0]) | ((v[1] == v[0]) & (i[1] < i[0]))
    return jnp.where(take, i[1], i[0])
